# baseline (speedup 1.0000x reference)
import functools
import jax
import jax.numpy as jnp
from jax import lax
from jax.experimental import pallas as pl
from jax.experimental.pallas import tpu as pltpu
from jax.experimental.pallas import tpu_sc as plsc

N = 10000
NPAD = 10240
E = 320000
NC, NS = 2, 16
NW = NC * NS
PT = 10240
EPAD = PT * NW
CH = 2048
NCHUNK = PT // CH
SUB = 128
NSUB = CH // SUB
RPT = NPAD // NS
H = 16


def _sc_deg_body(col2_hbm, ew2_hbm, out_hbm, col_v, ew_v, zb_v, deg_sh, sem):
    c = lax.axis_index("c")
    s = lax.axis_index("s")

    def zero(i, _):
        zb_v[pl.ds(i * 16, 16)] = jnp.zeros((16,), jnp.float32)
        return 0

    lax.fori_loop(0, RPT // 16, zero, 0)
    pltpu.sync_copy(zb_v, deg_sh.at[pl.ds(pl.multiple_of(s * RPT, 8), RPT)])
    plsc.subcore_barrier()

    base_r = (c * NS + s) * (PT // SUB)

    def chunk(k, _):
        rb = pl.multiple_of(base_r + k * NSUB, 8)
        pltpu.sync_copy(col2_hbm.at[pl.ds(rb, NSUB)], col_v)
        pltpu.sync_copy(ew2_hbm.at[pl.ds(rb, NSUB)], ew_v)
        hs = [
            pltpu.async_copy(ew_v.at[j], deg_sh.at[col_v.at[j]], sem, add=True)
            for j in range(NSUB)
        ]
        for h in hs:
            h.wait()
        return 0

    lax.fori_loop(0, NCHUNK, chunk, 0)
    plsc.subcore_barrier()
    pltpu.sync_copy(deg_sh.at[pl.ds(pl.multiple_of(s * RPT, 8), RPT)], zb_v)
    pltpu.sync_copy(zb_v, out_hbm.at[c, pl.ds(pl.multiple_of(s * RPT, 8), RPT)])


def _sc_edge_body(y_hbm, row_hbm, col2_hbm, ew_hbm, out_hbm,
                  idx_v, col_v, ew_v, rows_v, zb_v, agg_sh, sem, gsem):
    c = lax.axis_index("c")
    s = lax.axis_index("s")

    def zero(i, _):
        zb_v[i, :] = jnp.zeros((16,), jnp.float32)
        return 0

    lax.fori_loop(0, RPT, zero, 0)
    pltpu.sync_copy(zb_v, agg_sh.at[pl.ds(pl.multiple_of(s * RPT, 8), RPT)])
    plsc.subcore_barrier()

    base = (c * NS + s) * PT

    def chunk(k, _):
        eb = pl.multiple_of(base + k * CH, 8)
        pltpu.sync_copy(row_hbm.at[pl.ds(eb, CH)], idx_v)
        pltpu.sync_copy(col2_hbm.at[pl.ds(pl.multiple_of(eb // SUB, 8), NSUB)], col_v)
        pltpu.sync_copy(ew_hbm.at[pl.ds(eb, CH)], ew_v)
        pltpu.async_copy(y_hbm.at[idx_v], rows_v, gsem).wait()

        @plsc.parallel_loop(0, CH, 16, unroll=4)
        def mul(e0):
            w16 = ew_v[pl.ds(e0, 16)]
            for i in range(16):
                rows_v[e0 + i, :] = rows_v[e0 + i, :] * w16[i]
        hs = [
            pltpu.async_copy(rows_v.at[pl.ds(j * SUB, SUB)],
                             agg_sh.at[col_v.at[j]], sem, add=True)
            for j in range(NSUB)
        ]
        for h in hs:
            h.wait()
        return 0

    lax.fori_loop(0, NCHUNK, chunk, 0)
    plsc.subcore_barrier()
    pltpu.sync_copy(agg_sh.at[pl.ds(pl.multiple_of(s * RPT, 8), RPT)], zb_v)
    pltpu.sync_copy(zb_v, out_hbm.at[c, pl.ds(pl.multiple_of(s * RPT, 8), RPT)])


_sc_mesh = plsc.VectorSubcoreMesh(core_axis_name="c", subcore_axis_name="s")

_sc_deg = functools.partial(
    pl.kernel,
    out_type=jax.ShapeDtypeStruct((NC, NPAD), jnp.float32),
    mesh=_sc_mesh,
    compiler_params=pltpu.CompilerParams(use_tc_tiling_on_sc=False),
    scratch_types=[
        pltpu.VMEM((NSUB, SUB), jnp.int32),
        pltpu.VMEM((NSUB, SUB), jnp.float32),
        pltpu.VMEM((RPT,), jnp.float32),
        pltpu.VMEM_SHARED((NPAD,), jnp.float32),
        pltpu.SemaphoreType.DMA,
    ],
)(_sc_deg_body)

_sc_edge = functools.partial(
    pl.kernel,
    out_type=jax.ShapeDtypeStruct((NC, NPAD, H), jnp.float32),
    mesh=_sc_mesh,
    compiler_params=pltpu.CompilerParams(use_tc_tiling_on_sc=False),
    scratch_types=[
        pltpu.VMEM((CH,), jnp.int32),
        pltpu.VMEM((NSUB, SUB), jnp.int32),
        pltpu.VMEM((CH,), jnp.float32),
        pltpu.VMEM((CH, H), jnp.float32),
        pltpu.VMEM((RPT, H), jnp.float32),
        pltpu.VMEM_SHARED((NPAD, H), jnp.float32),
        pltpu.SemaphoreType.DMA,
        pltpu.SemaphoreType.DMA,
    ],
)(_sc_edge_body)


def _tc1_body(x_ref, w1_ref, d0_ref, d1_ref, y1_ref, dinv_ref):
    deg = d0_ref[...] + d1_ref[...] + 1.0
    dinv = jnp.where(deg > 0, lax.rsqrt(jnp.where(deg > 0, deg, 1.0)), 0.0)
    xw = jnp.dot(x_ref[...], w1_ref[...], preferred_element_type=jnp.float32)
    y1_ref[...] = dinv * xw
    dinv_ref[...] = dinv


def _tc2_body(a0_ref, a1_ref, y1_ref, dinv_ref, b1_ref, y2_ref):
    pre = dinv_ref[...] * (a0_ref[...] + a1_ref[...] + y1_ref[...]) + b1_ref[...]
    y2_ref[...] = dinv_ref[...] * jnp.maximum(pre, 0.0)


def _tc3_body(a0_ref, a1_ref, y2_ref, dinv_ref, w2_ref, b2_ref, out_ref):
    z = dinv_ref[...] * (a0_ref[...] + a1_ref[...] + y2_ref[...])
    out_ref[...] = (
        jnp.dot(z, w2_ref[...], preferred_element_type=jnp.float32) + b2_ref[...]
    )


def kernel(x, edge_index, edge_weight, W1, b1, W2, b2):
    row = edge_index[0]
    col = edge_index[1]
    pad = EPAD - E
    rowp = jnp.pad(row, (0, pad))
    colp = jnp.pad(col, (0, pad))
    ewp = jnp.pad(edge_weight, (0, pad))
    col2 = colp.reshape(EPAD // SUB, SUB)
    ew2 = ewp.reshape(EPAD // SUB, SUB)

    degp = _sc_deg(col2, ew2)
    d0 = degp[0, :N].reshape(N, 1)
    d1 = degp[1, :N].reshape(N, 1)

    y1, dinv = pl.pallas_call(
        _tc1_body,
        out_shape=[
            jax.ShapeDtypeStruct((N, H), jnp.float32),
            jax.ShapeDtypeStruct((N, 1), jnp.float32),
        ],
    )(x, W1, d0, d1)

    agg1 = _sc_edge(y1, rowp, col2, ewp)

    y2 = pl.pallas_call(
        _tc2_body,
        out_shape=jax.ShapeDtypeStruct((N, H), jnp.float32),
    )(agg1[0, :N], agg1[1, :N], y1, dinv, b1.reshape(1, H))

    agg2 = _sc_edge(y2, rowp, col2, ewp)

    out = pl.pallas_call(
        _tc3_body,
        out_shape=jax.ShapeDtypeStruct((N, 40), jnp.float32),
    )(agg2[0, :N], agg2[1, :N], y2, dinv, W2, b2.reshape(1, 40))
    return out


# speedup vs baseline: 1.0946x; 1.0946x over previous
import functools
import jax
import jax.numpy as jnp
from jax import lax
from jax.experimental import pallas as pl
from jax.experimental.pallas import tpu as pltpu
from jax.experimental.pallas import tpu_sc as plsc

N = 10000
NPAD = 10240
E = 320000
NC, NS = 2, 16
NW = NC * NS
PT = 10240
EPAD = PT * NW
CH = 2048
NCHUNK = PT // CH
SUB = 128
NSUB = CH // SUB
NB = 2
RPT = NPAD // NS
H = 16


def _sc_deg_body(col2_hbm, ew2_hbm, out_hbm, col_v, ew_v, zb_v, deg_sh, sem):
    c = lax.axis_index("c")
    s = lax.axis_index("s")

    def zero(i, _):
        zb_v[pl.ds(i * 16, 16)] = jnp.zeros((16,), jnp.float32)
        return 0

    lax.fori_loop(0, RPT // 16, zero, 0)
    pltpu.sync_copy(zb_v, deg_sh.at[pl.ds(pl.multiple_of(s * RPT, 8), RPT)])
    plsc.subcore_barrier()

    base_r = (c * NS + s) * (PT // SUB)

    def chunk(k, _):
        rb = pl.multiple_of(base_r + k * NSUB, 8)
        pltpu.sync_copy(col2_hbm.at[pl.ds(rb, NSUB)], col_v)
        pltpu.sync_copy(ew2_hbm.at[pl.ds(rb, NSUB)], ew_v)
        hs = [
            pltpu.async_copy(ew_v.at[j], deg_sh.at[col_v.at[j]], sem, add=True)
            for j in range(NSUB)
        ]
        for h in hs:
            h.wait()
        return 0

    lax.fori_loop(0, NCHUNK, chunk, 0)
    plsc.subcore_barrier()
    pltpu.sync_copy(deg_sh.at[pl.ds(pl.multiple_of(s * RPT, 8), RPT)], zb_v)
    pltpu.sync_copy(zb_v, out_hbm.at[c, pl.ds(pl.multiple_of(s * RPT, 8), RPT)])


def _sc_edge_body(y_hbm, row_hbm, col2_hbm, ew_hbm, out_hbm,
                  idx_v, col_v, ew_v, rows_v, zb_v, agg_sh,
                  isem, gsem, ssem):
    c = lax.axis_index("c")
    s = lax.axis_index("s")

    def zero(i, _):
        zb_v[i, :] = jnp.zeros((16,), jnp.float32)
        return 0

    lax.fori_loop(0, RPT, zero, 0)
    pltpu.sync_copy(zb_v, agg_sh.at[pl.ds(pl.multiple_of(s * RPT, 8), RPT)])
    plsc.subcore_barrier()

    base = (c * NS + s) * PT

    def start_load(k):
        b = k % NB
        eb = pl.multiple_of(base + k * CH, 8)
        return [
            pltpu.async_copy(row_hbm.at[pl.ds(eb, CH)], idx_v.at[b], isem[b]),
            pltpu.async_copy(
                col2_hbm.at[pl.ds(pl.multiple_of(eb // SUB, 8), NSUB)],
                col_v.at[b], isem[b]),
            pltpu.async_copy(ew_hbm.at[pl.ds(eb, CH)], ew_v.at[b], isem[b]),
        ]

    loads = [start_load(k) for k in range(NB)]
    scatters = [None] * NCHUNK
    for k in range(NCHUNK):
        b = k % NB
        for h in loads[k]:
            h.wait()
        if k >= NB:
            for h in scatters[k - NB]:
                h.wait()
        pltpu.async_copy(y_hbm.at[idx_v.at[b]], rows_v.at[b], gsem).wait()
        if k + NB < NCHUNK:
            loads.append(start_load(k + NB))

        @plsc.parallel_loop(0, CH, 16, unroll=4)
        def mul(e0):
            w16 = ew_v[b, pl.ds(e0, 16)]
            for i in range(16):
                rows_v[b, e0 + i, :] = rows_v[b, e0 + i, :] * w16[i]

        scatters[k] = [
            pltpu.async_copy(rows_v.at[b, pl.ds(j * SUB, SUB)],
                             agg_sh.at[col_v.at[b, j]], ssem, add=True)
            for j in range(NSUB)
        ]
    for k in range(NCHUNK - NB, NCHUNK):
        for h in scatters[k]:
            h.wait()
    plsc.subcore_barrier()
    pltpu.sync_copy(agg_sh.at[pl.ds(pl.multiple_of(s * RPT, 8), RPT)], zb_v)
    pltpu.sync_copy(zb_v, out_hbm.at[c, pl.ds(pl.multiple_of(s * RPT, 8), RPT)])


_sc_mesh = plsc.VectorSubcoreMesh(core_axis_name="c", subcore_axis_name="s")

_sc_deg = functools.partial(
    pl.kernel,
    out_type=jax.ShapeDtypeStruct((NC, NPAD), jnp.float32),
    mesh=_sc_mesh,
    compiler_params=pltpu.CompilerParams(use_tc_tiling_on_sc=False),
    scratch_types=[
        pltpu.VMEM((NSUB, SUB), jnp.int32),
        pltpu.VMEM((NSUB, SUB), jnp.float32),
        pltpu.VMEM((RPT,), jnp.float32),
        pltpu.VMEM_SHARED((NPAD,), jnp.float32),
        pltpu.SemaphoreType.DMA,
    ],
)(_sc_deg_body)

_sc_edge = functools.partial(
    pl.kernel,
    out_type=jax.ShapeDtypeStruct((NC, NPAD, H), jnp.float32),
    mesh=_sc_mesh,
    compiler_params=pltpu.CompilerParams(use_tc_tiling_on_sc=False),
    scratch_types=[
        pltpu.VMEM((NB, CH), jnp.int32),
        pltpu.VMEM((NB, NSUB, SUB), jnp.int32),
        pltpu.VMEM((NB, CH), jnp.float32),
        pltpu.VMEM((NB, CH, H), jnp.float32),
        pltpu.VMEM((RPT, H), jnp.float32),
        pltpu.VMEM_SHARED((NPAD, H), jnp.float32),
        [pltpu.SemaphoreType.DMA] * NB,
        pltpu.SemaphoreType.DMA,
        pltpu.SemaphoreType.DMA,
    ],
)(_sc_edge_body)


def _tc1_body(x_ref, w1_ref, d0_ref, d1_ref, y1_ref, dinv_ref):
    deg = d0_ref[...] + d1_ref[...] + 1.0
    dinv = jnp.where(deg > 0, lax.rsqrt(jnp.where(deg > 0, deg, 1.0)), 0.0)
    xw = jnp.dot(x_ref[...], w1_ref[...], preferred_element_type=jnp.float32)
    y1_ref[...] = dinv * xw
    dinv_ref[...] = dinv


def _tc2_body(a0_ref, a1_ref, y1_ref, dinv_ref, b1_ref, y2_ref):
    pre = dinv_ref[...] * (a0_ref[...] + a1_ref[...] + y1_ref[...]) + b1_ref[...]
    y2_ref[...] = dinv_ref[...] * jnp.maximum(pre, 0.0)


def _tc3_body(a0_ref, a1_ref, y2_ref, dinv_ref, w2_ref, b2_ref, out_ref):
    z = dinv_ref[...] * (a0_ref[...] + a1_ref[...] + y2_ref[...])
    out_ref[...] = (
        jnp.dot(z, w2_ref[...], preferred_element_type=jnp.float32) + b2_ref[...]
    )


def kernel(x, edge_index, edge_weight, W1, b1, W2, b2):
    row = edge_index[0]
    col = edge_index[1]
    pad = EPAD - E
    rowp = jnp.pad(row, (0, pad))
    colp = jnp.pad(col, (0, pad))
    ewp = jnp.pad(edge_weight, (0, pad))
    col2 = colp.reshape(EPAD // SUB, SUB)
    ew2 = ewp.reshape(EPAD // SUB, SUB)

    degp = _sc_deg(col2, ew2)
    d0 = degp[0, :N].reshape(N, 1)
    d1 = degp[1, :N].reshape(N, 1)

    y1, dinv = pl.pallas_call(
        _tc1_body,
        out_shape=[
            jax.ShapeDtypeStruct((N, H), jnp.float32),
            jax.ShapeDtypeStruct((N, 1), jnp.float32),
        ],
    )(x, W1, d0, d1)

    agg1 = _sc_edge(y1, rowp, col2, ewp)

    y2 = pl.pallas_call(
        _tc2_body,
        out_shape=jax.ShapeDtypeStruct((N, H), jnp.float32),
    )(agg1[0, :N], agg1[1, :N], y1, dinv, b1.reshape(1, H))

    agg2 = _sc_edge(y2, rowp, col2, ewp)

    out = pl.pallas_call(
        _tc3_body,
        out_shape=jax.ShapeDtypeStruct((N, 40), jnp.float32),
    )(agg2[0, :N], agg2[1, :N], y2, dinv, W2, b2.reshape(1, 40))
    return out


# speedup vs baseline: 1.1486x; 1.0494x over previous
import functools
import jax
import jax.numpy as jnp
from jax import lax
from jax.experimental import pallas as pl
from jax.experimental.pallas import tpu as pltpu
from jax.experimental.pallas import tpu_sc as plsc

N = 10000
NPAD = 10240
E = 320000
NC, NS = 2, 16
NW = NC * NS
PT = 10240
EPAD = PT * NW
CH = 2048
NCHUNK = PT // CH
SUB = 128
NSUB = CH // SUB
NB = 2
RPT = NPAD // NS
H = 16


def _sc_deg_body(col2_hbm, ew2_hbm, out_hbm, col_v, ew_v, zb_v, deg_sh, sem):
    c = lax.axis_index("c")
    s = lax.axis_index("s")

    def zero(i, _):
        zb_v[pl.ds(i * 16, 16)] = jnp.zeros((16,), jnp.float32)
        return 0

    lax.fori_loop(0, RPT // 16, zero, 0)
    pltpu.sync_copy(zb_v, deg_sh.at[pl.ds(pl.multiple_of(s * RPT, 8), RPT)])
    plsc.subcore_barrier()

    base_r = (c * NS + s) * (PT // SUB)

    def chunk(k, _):
        rb = pl.multiple_of(base_r + k * NSUB, 8)
        pltpu.sync_copy(col2_hbm.at[pl.ds(rb, NSUB)], col_v)
        pltpu.sync_copy(ew2_hbm.at[pl.ds(rb, NSUB)], ew_v)
        hs = [
            pltpu.async_copy(ew_v.at[j], deg_sh.at[col_v.at[j]], sem, add=True)
            for j in range(NSUB)
        ]
        for h in hs:
            h.wait()
        return 0

    lax.fori_loop(0, NCHUNK, chunk, 0)
    plsc.subcore_barrier()
    pltpu.sync_copy(deg_sh.at[pl.ds(pl.multiple_of(s * RPT, 8), RPT)], zb_v)
    pltpu.sync_copy(zb_v, out_hbm.at[c, pl.ds(pl.multiple_of(s * RPT, 8), RPT)])


def _sc_edge_body(y_hbm, row_hbm, col2_hbm, ew_hbm, out_hbm,
                  idx_v, col_v, ew_v, rows_v, zb_v, agg_sh,
                  isem, gsem, ssem):
    c = lax.axis_index("c")
    s = lax.axis_index("s")

    def zero(i, _):
        zb_v[i, :] = jnp.zeros((16,), jnp.float32)
        return 0

    lax.fori_loop(0, RPT, zero, 0)
    pltpu.sync_copy(zb_v, agg_sh.at[pl.ds(pl.multiple_of(s * RPT, 8), RPT)])
    plsc.subcore_barrier()

    base = (c * NS + s) * PT

    def start_load(k):
        b = k % 3
        eb = pl.multiple_of(base + k * CH, 8)
        return [
            pltpu.async_copy(row_hbm.at[pl.ds(eb, CH)], idx_v.at[b], isem[b]),
            pltpu.async_copy(
                col2_hbm.at[pl.ds(pl.multiple_of(eb // SUB, 8), NSUB)],
                col_v.at[b], isem[b]),
            pltpu.async_copy(ew_hbm.at[pl.ds(eb, CH)], ew_v.at[b], isem[b]),
        ]

    def start_gather(k):
        return pltpu.async_copy(y_hbm.at[idx_v.at[k % 3]],
                                rows_v.at[k % 2], gsem)

    loads = {k: start_load(k) for k in range(min(2, NCHUNK))}
    for h in loads[0]:
        h.wait()
    gathers = {0: start_gather(0)}
    scatters = {}
    for k in range(NCHUNK):
        bi = k % 3
        br = k % 2
        gathers[k].wait()
        if k >= 1:
            for h in scatters[k - 1]:
                h.wait()
        if k + 2 < NCHUNK:
            loads[k + 2] = start_load(k + 2)
        if k + 1 < NCHUNK:
            for h in loads[k + 1]:
                h.wait()
            gathers[k + 1] = start_gather(k + 1)

        @plsc.parallel_loop(0, CH, 16, unroll=4)
        def mul(e0):
            w16 = ew_v[bi, pl.ds(e0, 16)]
            for i in range(16):
                rows_v[br, e0 + i, :] = rows_v[br, e0 + i, :] * w16[i]

        scatters[k] = [
            pltpu.async_copy(rows_v.at[br, pl.ds(j * SUB, SUB)],
                             agg_sh.at[col_v.at[bi, j]], ssem, add=True)
            for j in range(NSUB)
        ]
    for h in scatters[NCHUNK - 1]:
        h.wait()
    plsc.subcore_barrier()
    pltpu.sync_copy(agg_sh.at[pl.ds(pl.multiple_of(s * RPT, 8), RPT)], zb_v)
    pltpu.sync_copy(zb_v, out_hbm.at[c, pl.ds(pl.multiple_of(s * RPT, 8), RPT)])


_sc_mesh = plsc.VectorSubcoreMesh(core_axis_name="c", subcore_axis_name="s")

_sc_deg = functools.partial(
    pl.kernel,
    out_type=jax.ShapeDtypeStruct((NC, NPAD), jnp.float32),
    mesh=_sc_mesh,
    compiler_params=pltpu.CompilerParams(use_tc_tiling_on_sc=False),
    scratch_types=[
        pltpu.VMEM((NSUB, SUB), jnp.int32),
        pltpu.VMEM((NSUB, SUB), jnp.float32),
        pltpu.VMEM((RPT,), jnp.float32),
        pltpu.VMEM_SHARED((NPAD,), jnp.float32),
        pltpu.SemaphoreType.DMA,
    ],
)(_sc_deg_body)

_sc_edge = functools.partial(
    pl.kernel,
    out_type=jax.ShapeDtypeStruct((NC, NPAD, H), jnp.float32),
    mesh=_sc_mesh,
    compiler_params=pltpu.CompilerParams(use_tc_tiling_on_sc=False),
    scratch_types=[
        pltpu.VMEM((3, CH), jnp.int32),
        pltpu.VMEM((3, NSUB, SUB), jnp.int32),
        pltpu.VMEM((3, CH), jnp.float32),
        pltpu.VMEM((NB, CH, H), jnp.float32),
        pltpu.VMEM((RPT, H), jnp.float32),
        pltpu.VMEM_SHARED((NPAD, H), jnp.float32),
        [pltpu.SemaphoreType.DMA] * 3,
        pltpu.SemaphoreType.DMA,
        pltpu.SemaphoreType.DMA,
    ],
)(_sc_edge_body)


def _tc1_body(x_ref, w1_ref, d0_ref, d1_ref, y1_ref, dinv_ref):
    deg = d0_ref[...] + d1_ref[...] + 1.0
    dinv = jnp.where(deg > 0, lax.rsqrt(jnp.where(deg > 0, deg, 1.0)), 0.0)
    xw = jnp.dot(x_ref[...], w1_ref[...], preferred_element_type=jnp.float32)
    y1_ref[...] = dinv * xw
    dinv_ref[...] = dinv


def _tc2_body(a0_ref, a1_ref, y1_ref, dinv_ref, b1_ref, y2_ref):
    pre = dinv_ref[...] * (a0_ref[...] + a1_ref[...] + y1_ref[...]) + b1_ref[...]
    y2_ref[...] = dinv_ref[...] * jnp.maximum(pre, 0.0)


def _tc3_body(a0_ref, a1_ref, y2_ref, dinv_ref, w2_ref, b2_ref, out_ref):
    z = dinv_ref[...] * (a0_ref[...] + a1_ref[...] + y2_ref[...])
    out_ref[...] = (
        jnp.dot(z, w2_ref[...], preferred_element_type=jnp.float32) + b2_ref[...]
    )


def kernel(x, edge_index, edge_weight, W1, b1, W2, b2):
    row = edge_index[0]
    col = edge_index[1]
    pad = EPAD - E
    rowp = jnp.pad(row, (0, pad))
    colp = jnp.pad(col, (0, pad))
    ewp = jnp.pad(edge_weight, (0, pad))
    col2 = colp.reshape(EPAD // SUB, SUB)
    ew2 = ewp.reshape(EPAD // SUB, SUB)

    degp = _sc_deg(col2, ew2)
    d0 = degp[0, :N].reshape(N, 1)
    d1 = degp[1, :N].reshape(N, 1)

    y1, dinv = pl.pallas_call(
        _tc1_body,
        out_shape=[
            jax.ShapeDtypeStruct((N, H), jnp.float32),
            jax.ShapeDtypeStruct((N, 1), jnp.float32),
        ],
    )(x, W1, d0, d1)

    agg1 = _sc_edge(y1, rowp, col2, ewp)

    y2 = pl.pallas_call(
        _tc2_body,
        out_shape=jax.ShapeDtypeStruct((N, H), jnp.float32),
    )(agg1[0, :N], agg1[1, :N], y1, dinv, b1.reshape(1, H))

    agg2 = _sc_edge(y2, rowp, col2, ewp)

    out = pl.pallas_call(
        _tc3_body,
        out_shape=jax.ShapeDtypeStruct((N, 40), jnp.float32),
    )(agg2[0, :N], agg2[1, :N], y2, dinv, W2, b2.reshape(1, 40))
    return out


# speedup vs baseline: 1.5356x; 1.3370x over previous
import functools
import jax
import jax.numpy as jnp
from jax import lax
from jax.experimental import pallas as pl
from jax.experimental.pallas import tpu as pltpu
from jax.experimental.pallas import tpu_sc as plsc

N = 10000
NPAD = 10240
E = 320000
NC, NS = 2, 16
NW = NC * NS
PT = 10240
EPAD = PT * NW
CH = 1024
NCHUNK = PT // CH
DCH = 2 * PT // CH
SUB = 256
NSUB = CH // SUB
NB = 2
RPT = NPAD // NS
H = 16


def _agg_pipeline(c, s, row_hbm, col2_hbm, ew2_hbm, y_sh, agg_sh,
                  idx_v, col_v, ew_v, rows_v, isem, gsem, ssem):
    base = (c * NS + s) * PT

    def start_load(k):
        b = k % 3
        eb = pl.multiple_of(base + k * CH, 8)
        rb = pl.multiple_of(eb // SUB, 4)
        return [
            pltpu.async_copy(row_hbm.at[pl.ds(eb, CH)], idx_v.at[b], isem[b]),
            pltpu.async_copy(col2_hbm.at[pl.ds(rb, NSUB)], col_v.at[b],
                             isem[b]),
            pltpu.async_copy(ew2_hbm.at[pl.ds(rb, NSUB)], ew_v.at[b],
                             isem[b]),
        ]

    def start_gather(k):
        return pltpu.async_copy(y_sh.at[idx_v.at[k % 3]],
                                rows_v.at[k % 2], gsem)

    loads = {k: start_load(k) for k in range(min(2, NCHUNK))}
    for h in loads[0]:
        h.wait()
    gathers = {0: start_gather(0)}
    scatters = {}
    for k in range(NCHUNK):
        bi = k % 3
        br = k % 2
        gathers[k].wait()
        if k >= 1:
            for h in scatters[k - 1]:
                h.wait()
        if k + 2 < NCHUNK:
            loads[k + 2] = start_load(k + 2)
        if k + 1 < NCHUNK:
            for h in loads[k + 1]:
                h.wait()
            gathers[k + 1] = start_gather(k + 1)

        @plsc.parallel_loop(0, CH, 16, unroll=4)
        def mul(e0):
            w16 = ew_v[bi, e0 // SUB, pl.ds(lax.rem(e0, SUB), 16)]
            for i in range(16):
                rows_v[br, e0 + i, :] = rows_v[br, e0 + i, :] * w16[i]

        scatters[k] = [
            pltpu.async_copy(rows_v.at[br, pl.ds(j * SUB, SUB)],
                             agg_sh.at[col_v.at[bi, j]], ssem, add=True)
            for j in range(NSUB)
        ]
    for h in scatters[NCHUNK - 1]:
        h.wait()


def _zero_rows(zb_v):
    def zero(i, _):
        zb_v[i, :] = jnp.zeros((16,), jnp.float32)
        return 0

    lax.fori_loop(0, RPT, zero, 0)


def _sc_a_body(xw_hbm, row_hbm, col2_hbm, ew2_hbm,
               agg_out, dinv_out, y1_out,
               idx_v, col_v, ew_v, dew_v, rows_v, zb_v, dg_v,
               deg_sh, agg_sh, y_sh, isem, gsem, ssem):
    c = lax.axis_index("c")
    s = lax.axis_index("s")
    nb8 = pl.multiple_of(s * RPT, 8)

    _zero_rows(zb_v)
    pltpu.sync_copy(zb_v, agg_sh.at[pl.ds(nb8, RPT)])

    def zero1(i, _):
        dg_v[pl.ds(i * 16, 16)] = jnp.zeros((16,), jnp.float32)
        return 0

    lax.fori_loop(0, RPT // 16, zero1, 0)
    pltpu.sync_copy(dg_v, deg_sh.at[pl.ds(nb8, RPT)])
    plsc.subcore_barrier()

    dbase_r = s * (2 * PT // SUB)

    def dstart_load(k):
        b = k % 3
        rb = pl.multiple_of(dbase_r + k * NSUB, 4)
        return [
            pltpu.async_copy(col2_hbm.at[pl.ds(rb, NSUB)], col_v.at[b],
                             isem[b]),
            pltpu.async_copy(ew2_hbm.at[pl.ds(rb, NSUB)], dew_v.at[b],
                             isem[b]),
        ]

    dloads = {k: dstart_load(k) for k in range(min(2, DCH))}
    dscat = {}
    for k in range(DCH):
        b = k % 3
        for h in dloads[k]:
            h.wait()
        if k >= 1:
            for h in dscat[k - 1]:
                h.wait()
        if k + 2 < DCH:
            dloads[k + 2] = dstart_load(k + 2)
        dscat[k] = [
            pltpu.async_copy(dew_v.at[b, j], deg_sh.at[col_v.at[b, j]],
                             ssem, add=True)
            for j in range(NSUB)
        ]
    for h in dscat[DCH - 1]:
        h.wait()
    plsc.subcore_barrier()

    pltpu.sync_copy(deg_sh.at[pl.ds(nb8, RPT)], dg_v)

    @plsc.parallel_loop(0, RPT, 16, unroll=2)
    def dinv_loop(n0):
        d = dg_v[pl.ds(n0, 16)] + 1.0
        i = jnp.full((16,), 0x5F3759DF, jnp.int32) - lax.shift_right_logical(
            plsc.bitcast(d, jnp.int32), 1)
        xx = plsc.bitcast(i, jnp.float32)
        for _ in range(3):
            xx = xx * (1.5 - 0.5 * d * xx * xx)
        dg_v[pl.ds(n0, 16)] = xx

    pltpu.sync_copy(xw_hbm.at[pl.ds(nb8, RPT)], zb_v)

    @plsc.parallel_loop(0, RPT, 16, unroll=2)
    def scale(n0):
        d16 = dg_v[pl.ds(n0, 16)]
        for i in range(16):
            zb_v[n0 + i, :] = zb_v[n0 + i, :] * d16[i]

    pltpu.sync_copy(zb_v, y_sh.at[pl.ds(nb8, RPT)])

    @pl.when(c == 0)
    def _():
        pltpu.sync_copy(zb_v, y1_out.at[pl.ds(nb8, RPT)])
        pltpu.sync_copy(dg_v, dinv_out.at[pl.ds(nb8, RPT)])

    plsc.subcore_barrier()

    _agg_pipeline(c, s, row_hbm, col2_hbm, ew2_hbm, y_sh, agg_sh,
                  idx_v, col_v, ew_v, rows_v, isem, gsem, ssem)
    plsc.subcore_barrier()
    pltpu.sync_copy(agg_sh.at[pl.ds(nb8, RPT)], zb_v)
    pltpu.sync_copy(zb_v, agg_out.at[c, pl.ds(nb8, RPT)])


def _sc_b_body(a0_hbm, a1_hbm, y1_hbm, dinv_hbm, b1_hbm,
               row_hbm, col2_hbm, ew2_hbm,
               agg_out, y2_out,
               idx_v, col_v, ew_v, rows_v, zb_v, sa_v, sb_v, dg_v, b1_v,
               agg_sh, y_sh, isem, gsem, ssem):
    c = lax.axis_index("c")
    s = lax.axis_index("s")
    nb8 = pl.multiple_of(s * RPT, 8)

    _zero_rows(zb_v)
    pltpu.sync_copy(zb_v, agg_sh.at[pl.ds(nb8, RPT)])
    pltpu.sync_copy(b1_hbm, b1_v)
    pltpu.sync_copy(a0_hbm.at[pl.ds(nb8, RPT)], sa_v)
    pltpu.sync_copy(a1_hbm.at[pl.ds(nb8, RPT)], sb_v)
    pltpu.sync_copy(y1_hbm.at[pl.ds(nb8, RPT)], zb_v)
    pltpu.sync_copy(dinv_hbm.at[pl.ds(nb8, RPT)], dg_v)
    b1vec = b1_v[...]

    @plsc.parallel_loop(0, RPT, 16, unroll=2)
    def y2loop(n0):
        d16 = dg_v[pl.ds(n0, 16)]
        for i in range(16):
            r = sa_v[n0 + i, :] + sb_v[n0 + i, :] + zb_v[n0 + i, :]
            t = jnp.maximum(r * d16[i] + b1vec, 0.0)
            zb_v[n0 + i, :] = t * d16[i]

    pltpu.sync_copy(zb_v, y_sh.at[pl.ds(nb8, RPT)])

    @pl.when(c == 0)
    def _():
        pltpu.sync_copy(zb_v, y2_out.at[pl.ds(nb8, RPT)])

    plsc.subcore_barrier()

    _agg_pipeline(c, s, row_hbm, col2_hbm, ew2_hbm, y_sh, agg_sh,
                  idx_v, col_v, ew_v, rows_v, isem, gsem, ssem)
    plsc.subcore_barrier()
    pltpu.sync_copy(agg_sh.at[pl.ds(nb8, RPT)], zb_v)
    pltpu.sync_copy(zb_v, agg_out.at[c, pl.ds(nb8, RPT)])


_sc_mesh = plsc.VectorSubcoreMesh(core_axis_name="c", subcore_axis_name="s")

_sc_a = functools.partial(
    pl.kernel,
    out_type=[
        jax.ShapeDtypeStruct((NC, NPAD, H), jnp.float32),
        jax.ShapeDtypeStruct((NPAD,), jnp.float32),
        jax.ShapeDtypeStruct((NPAD, H), jnp.float32),
    ],
    mesh=_sc_mesh,
    compiler_params=pltpu.CompilerParams(use_tc_tiling_on_sc=False,
                                         needs_layout_passes=False),
    scratch_types=[
        pltpu.VMEM((3, CH), jnp.int32),
        pltpu.VMEM((3, NSUB, SUB), jnp.int32),
        pltpu.VMEM((3, NSUB, SUB), jnp.float32),
        pltpu.VMEM((3, NSUB, SUB), jnp.float32),
        pltpu.VMEM((NB, CH, H), jnp.float32),
        pltpu.VMEM((RPT, H), jnp.float32),
        pltpu.VMEM((RPT,), jnp.float32),
        pltpu.VMEM_SHARED((NPAD,), jnp.float32),
        pltpu.VMEM_SHARED((NPAD, H), jnp.float32),
        pltpu.VMEM_SHARED((NPAD, H), jnp.float32),
        [pltpu.SemaphoreType.DMA] * 3,
        pltpu.SemaphoreType.DMA,
        pltpu.SemaphoreType.DMA,
    ],
)(_sc_a_body)

_sc_b = functools.partial(
    pl.kernel,
    out_type=[
        jax.ShapeDtypeStruct((NC, NPAD, H), jnp.float32),
        jax.ShapeDtypeStruct((NPAD, H), jnp.float32),
    ],
    mesh=_sc_mesh,
    compiler_params=pltpu.CompilerParams(use_tc_tiling_on_sc=False,
                                         needs_layout_passes=False),
    scratch_types=[
        pltpu.VMEM((3, CH), jnp.int32),
        pltpu.VMEM((3, NSUB, SUB), jnp.int32),
        pltpu.VMEM((3, NSUB, SUB), jnp.float32),
        pltpu.VMEM((NB, CH, H), jnp.float32),
        pltpu.VMEM((RPT, H), jnp.float32),
        pltpu.VMEM((RPT, H), jnp.float32),
        pltpu.VMEM((RPT, H), jnp.float32),
        pltpu.VMEM((RPT,), jnp.float32),
        pltpu.VMEM((H,), jnp.float32),
        pltpu.VMEM_SHARED((NPAD, H), jnp.float32),
        pltpu.VMEM_SHARED((NPAD, H), jnp.float32),
        [pltpu.SemaphoreType.DMA] * 3,
        pltpu.SemaphoreType.DMA,
        pltpu.SemaphoreType.DMA,
    ],
)(_sc_b_body)


def _tc1_body(x_ref, w1_ref, xw_ref):
    xw_ref[pl.ds(0, N), :] = jnp.dot(
        x_ref[...], w1_ref[...], preferred_element_type=jnp.float32)


def _tc3_body(a0_ref, a1_ref, y2_ref, dinv_ref, w2_ref, b2_ref, out_ref):
    z = dinv_ref[...] * (a0_ref[...] + a1_ref[...] + y2_ref[...])
    out_ref[...] = (
        jnp.dot(z, w2_ref[...], preferred_element_type=jnp.float32) + b2_ref[...]
    )


def kernel(x, edge_index, edge_weight, W1, b1, W2, b2):
    row = edge_index[0]
    col = edge_index[1]
    pad = EPAD - E
    rowp = jnp.pad(row, (0, pad))
    colp = jnp.pad(col, (0, pad))
    ewp = jnp.pad(edge_weight, (0, pad))
    col2 = colp.reshape(EPAD // SUB, SUB)
    ew2 = ewp.reshape(EPAD // SUB, SUB)

    xw = pl.pallas_call(
        _tc1_body,
        out_shape=jax.ShapeDtypeStruct((NPAD, H), jnp.float32),
    )(x, W1)

    agg1, dinv_all, y1_all = _sc_a(xw, rowp, col2, ew2)
    agg2, y2_all = _sc_b(agg1[0], agg1[1], y1_all, dinv_all, b1,
                         rowp, col2, ew2)

    out = pl.pallas_call(
        _tc3_body,
        out_shape=jax.ShapeDtypeStruct((N, 40), jnp.float32),
    )(agg2[0, :N], agg2[1, :N], y2_all[:N], dinv_all[:N].reshape(N, 1),
      W2, b2.reshape(1, 40))
    return out
